# baseline (speedup 1.0000x reference)
import functools

import jax
import jax.numpy as jnp
from jax import lax
from jax.experimental import pallas as pl
from jax.experimental.pallas import tpu as pltpu
from jax.experimental.pallas import tpu_sc as plsc

_BATCH = 16384
_DIM = 64
_NC = 2
_NS = 16
_NW = _NC * _NS
_BPW = _BATCH // _NW
_CHUNK = 128
_NCHUNK = _BPW // _CHUNK
_GRP = 16


def _softplus(x):
    v = jnp.exp(-jnp.abs(x))
    w = v / (v + 2.0)
    w2 = w * w
    log1p = 2.0 * w * (1.0 + w2 * (1.0 / 3.0 + w2 * (0.2 + w2 * (1.0 / 7.0 + w2 / 9.0))))
    return jnp.maximum(x, 0.0) + log1p


def _sc_body(h_idx, r_idx, t_idx, ent_emb, rel_emb, out_hbm,
             idx_v, h_rows, r_rows, t_rows, scores, sem):
    wid = lax.axis_index("s") * _NC + lax.axis_index("c")
    cbase = wid * _NCHUNK

    pltpu.sync_copy(h_idx.at[pl.ds(cbase, _NCHUNK)], idx_v.at[0])
    pltpu.sync_copy(r_idx.at[pl.ds(cbase, _NCHUNK)], idx_v.at[1])
    pltpu.sync_copy(t_idx.at[pl.ds(cbase, _NCHUNK)], idx_v.at[2])

    copies = []
    for c in range(_NCHUNK):
        dst = pl.ds(c * _CHUNK, _CHUNK)
        copies.append(pltpu.async_copy(ent_emb.at[idx_v.at[0, c]], h_rows.at[dst], sem))
        copies.append(pltpu.async_copy(rel_emb.at[idx_v.at[1, c]], r_rows.at[dst], sem))
        copies.append(pltpu.async_copy(ent_emb.at[idx_v.at[2, c]], t_rows.at[dst], sem))

    lane = lax.iota(jnp.int32, _GRP)
    lane_masks = [lane == k for k in range(_GRP)]
    zeros = jnp.zeros((_GRP,), jnp.float32)

    for c in range(_NCHUNK):
        for cp in copies[3 * c:3 * c + 3]:
            cp.wait()

        def dot_body(g, carry):
            out = zeros
            for k in range(_GRP):
                j = g * _GRP + k
                acc = (h_rows[j, pl.ds(0, 16)] * r_rows[j, pl.ds(0, 16)]
                       * t_rows[j, pl.ds(0, 16)])
                for d in range(1, _DIM // 16):
                    acc = acc + (h_rows[j, pl.ds(d * 16, 16)]
                                 * r_rows[j, pl.ds(d * 16, 16)]
                                 * t_rows[j, pl.ds(d * 16, 16)])
                s = jnp.sum(acc)
                out = jnp.where(lane_masks[k], s, out)
            scores[pl.ds(g * _GRP, _GRP)] = _softplus(out)
            return carry

        lax.fori_loop(c * (_CHUNK // _GRP), (c + 1) * (_CHUNK // _GRP),
                      dot_body, 0)

    pltpu.sync_copy(scores, out_hbm.at[pl.ds(wid * _BPW, _BPW)])


@jax.jit
def _focus_e_sc(h_idx, r_idx, t_idx, ent_emb, rel_emb):
    mesh = plsc.VectorSubcoreMesh(core_axis_name="c", subcore_axis_name="s")
    kern = functools.partial(
        pl.kernel,
        mesh=mesh,
        compiler_params=pltpu.CompilerParams(
            needs_layout_passes=False, use_tc_tiling_on_sc=False),
        out_type=jax.ShapeDtypeStruct((_BATCH,), jnp.float32),
        scratch_types=[
            pltpu.VMEM((3, _NCHUNK, _CHUNK), jnp.int32),
            pltpu.VMEM((_BPW, _DIM), jnp.float32),
            pltpu.VMEM((_BPW, _DIM), jnp.float32),
            pltpu.VMEM((_BPW, _DIM), jnp.float32),
            pltpu.VMEM((_BPW,), jnp.float32),
            pltpu.SemaphoreType.DMA,
        ],
    )(_sc_body)
    return kern(h_idx, r_idx, t_idx, ent_emb, rel_emb)


def kernel(triples, ent_emb, rel_emb):
    idx = triples.astype(jnp.int32)
    h_idx = idx[:, 0].reshape(_BATCH // _CHUNK, _CHUNK)
    r_idx = idx[:, 1].reshape(_BATCH // _CHUNK, _CHUNK)
    t_idx = idx[:, 2].reshape(_BATCH // _CHUNK, _CHUNK)
    return _focus_e_sc(h_idx, r_idx, t_idx, ent_emb, rel_emb)


# speedup vs baseline: 1.5601x; 1.5601x over previous
import functools

import jax
import jax.numpy as jnp
from jax import lax
from jax.experimental import pallas as pl
from jax.experimental.pallas import tpu as pltpu
from jax.experimental.pallas import tpu_sc as plsc

_BATCH = 16384
_DIM = 64
_NC = 2
_NS = 16
_NW = _NC * _NS
_BPW = _BATCH // _NW
_GRP = 16
_IGRP = 8
_CHUNK = 128
_NCHUNK = _BPW // _CHUNK


def _softplus(x):
    v = jnp.exp(-jnp.abs(x))
    w = v / (v + 2.0)
    w2 = w * w
    log1p = 2.0 * w * (1.0 + w2 * (1.0 / 3.0 + w2 * (0.2 + w2 * (1.0 / 7.0 + w2 / 9.0))))
    return jnp.maximum(x, 0.0) + log1p


def _sc_body(h_idx, r_idx, t_idx, ent_emb, rel_emb, out_hbm,
             idx_h, idx_r, idx_t,
             h0, r0, t0, h1, r1, t1,
             scores, sem0, sem1):
    wid = lax.axis_index("s") * _NC + lax.axis_index("c")
    base = wid * _BPW

    pltpu.sync_copy(h_idx.at[pl.ds(base, _BPW)], idx_h.at[pl.ds(0, _BPW)])
    pltpu.sync_copy(r_idx.at[pl.ds(base, _BPW)], idx_r.at[pl.ds(0, _BPW)])
    pltpu.sync_copy(t_idx.at[pl.ds(base, _BPW)], idx_t.at[pl.ds(0, _BPW)])

    bufs = ((h0, r0, t0), (h1, r1, t1))
    sems = (sem0, sem1)
    lane = lax.iota(jnp.int32, _GRP)
    lane_masks = [lane == k for k in range(_GRP)]
    zeros = jnp.zeros((_GRP,), jnp.float32)

    def issue(c, slot):
        hb, rb, tb = bufs[slot]
        sem = sems[slot]

        def issue_body(g, carry):
            sl = pl.ds(g * _IGRP, _GRP)
            hv, rv, tv = idx_h[sl], idx_r[sl], idx_t[sl]
            jb = g * _IGRP - c * _CHUNK
            for k in range(_IGRP):
                j = jb + k
                pltpu.async_copy(ent_emb.at[hv[k]], hb.at[j], sem)
                pltpu.async_copy(rel_emb.at[rv[k]], rb.at[j], sem)
                pltpu.async_copy(ent_emb.at[tv[k]], tb.at[j], sem)
            return carry

        cg = _CHUNK // _IGRP
        lax.fori_loop(c * cg, (c + 1) * cg, issue_body, 0)

    def drain(slot):
        hb, rb, tb = bufs[slot]
        sem = sems[slot]
        src = ent_emb.at[pl.ds(0, _CHUNK)]
        pltpu.make_async_copy(src, hb, sem).wait()
        pltpu.make_async_copy(src, rb, sem).wait()
        pltpu.make_async_copy(src, tb, sem).wait()

    def compute(c, slot):
        hb, rb, tb = bufs[slot]

        def dot_body(g, carry):
            jb = g * _GRP - c * _CHUNK
            out = zeros
            for k in range(_GRP):
                j = jb + k
                acc = (hb[j, pl.ds(0, 16)] * rb[j, pl.ds(0, 16)]
                       * tb[j, pl.ds(0, 16)])
                for d in range(1, _DIM // 16):
                    acc = acc + (hb[j, pl.ds(d * 16, 16)]
                                 * rb[j, pl.ds(d * 16, 16)]
                                 * tb[j, pl.ds(d * 16, 16)])
                s = jnp.sum(acc)
                out = jnp.where(lane_masks[k], s, out)
            scores[pl.ds(g * _GRP, _GRP)] = _softplus(out)
            return carry

        cg = _CHUNK // _GRP
        lax.fori_loop(c * cg, (c + 1) * cg, dot_body, 0)

    issue(0, 0)
    for c in range(_NCHUNK):
        if c + 1 < _NCHUNK:
            issue(c + 1, (c + 1) & 1)
        drain(c & 1)
        compute(c, c & 1)

    pltpu.sync_copy(scores, out_hbm.at[pl.ds(base, _BPW)])


@jax.jit
def _focus_e_sc(h_idx, r_idx, t_idx, ent_emb, rel_emb):
    mesh = plsc.VectorSubcoreMesh(core_axis_name="c", subcore_axis_name="s")
    rows = pltpu.VMEM((_CHUNK, _DIM), jnp.float32)
    kern = functools.partial(
        pl.kernel,
        mesh=mesh,
        compiler_params=pltpu.CompilerParams(
            needs_layout_passes=False, use_tc_tiling_on_sc=True),
        out_type=jax.ShapeDtypeStruct((_BATCH,), jnp.float32),
        scratch_types=[
            pltpu.VMEM((_BPW + _GRP,), jnp.int32),
            pltpu.VMEM((_BPW + _GRP,), jnp.int32),
            pltpu.VMEM((_BPW + _GRP,), jnp.int32),
            rows, rows, rows, rows, rows, rows,
            pltpu.VMEM((_BPW,), jnp.float32),
            pltpu.SemaphoreType.DMA,
            pltpu.SemaphoreType.DMA,
        ],
    )(_sc_body)
    return kern(h_idx, r_idx, t_idx, ent_emb, rel_emb)


def kernel(triples, ent_emb, rel_emb):
    idx = triples.astype(jnp.int32)
    return _focus_e_sc(idx[:, 0], idx[:, 1], idx[:, 2], ent_emb, rel_emb)
